# baseline (speedup 1.0000x reference)
import functools

import jax
import jax.numpy as jnp
from jax import lax
from jax.experimental import pallas as pl
from jax.experimental.pallas import tpu as pltpu
from jax.experimental.pallas import tpu_sc as plsc

_NC = 2
_NS = 16
_L = 16
_NW = _NC * _NS

_B = 16384
_D = 32
_BPW = _B // _NW
_CHUNK = 128
_NCH = _BPW // _CHUNK


def _mf_body(u_hbm, v_hbm, ue_hbm, ve_hbm, out_hbm,
             ui_v, vi_v, uq_v, vq_v, ue_b, ve_b, o_v, semu, semv):
    wid = lax.axis_index("s") * _NC + lax.axis_index("c")
    base = wid * _BPW

    pltpu.sync_copy(u_hbm.at[pl.ds(base, _BPW)], ui_v)
    pltpu.sync_copy(v_hbm.at[pl.ds(base, _BPW)], vi_v)

    def quarter(i, carry):
        sl = pl.ds(i * _L, _L)
        uq_v[sl] = lax.shift_right_logical(ui_v[sl], 2)
        vq_v[sl] = lax.shift_right_logical(vi_v[sl], 2)
        return carry

    lax.fori_loop(0, _BPW // _L, quarter, 0)

    iota = lax.iota(jnp.int32, _L)

    for c in range(_NCH):
        csl = pl.ds(c * _CHUNK, _CHUNK)
        cu = pltpu.async_copy(ue_hbm.at[uq_v.at[csl]], ue_b, semu)
        cv = pltpu.async_copy(ve_hbm.at[vq_v.at[csl]], ve_b, semv)
        cu.wait()
        cv.wait()

        def group_body(g, carry, c=c):
            row0 = g * _L
            rows = row0 + iota
            ucols = (ui_v[pl.ds(c * _CHUNK + row0, _L)] & 3) * _D
            vcols = (vi_v[pl.ds(c * _CHUNK + row0, _L)] & 3) * _D

            def d_body(d, acc):
                a = plsc.load_gather(ue_b, [rows, ucols + d])
                b = plsc.load_gather(ve_b, [rows, vcols + d])
                return acc + a * b

            acc = lax.fori_loop(0, _D, d_body, jnp.zeros((_L,), jnp.float32))
            o_v[pl.ds(c * _CHUNK + row0, _L)] = (
                4.0 / (1.0 + jnp.exp(-acc)) + 1.0
            )
            return carry

        lax.fori_loop(0, _CHUNK // _L, group_body, 0)

    pltpu.sync_copy(o_v, out_hbm.at[pl.ds(base, _BPW)])


def kernel(u, v, user_emb, movie_emb):
    ue2 = user_emb.reshape(-1, 4 * _D)
    ve2 = movie_emb.reshape(-1, 4 * _D)
    mesh = plsc.VectorSubcoreMesh(core_axis_name="c", subcore_axis_name="s")
    run = functools.partial(
        pl.kernel,
        out_type=jax.ShapeDtypeStruct((_B,), jnp.float32),
        mesh=mesh,
        compiler_params=pltpu.CompilerParams(
            needs_layout_passes=False, use_tc_tiling_on_sc=False
        ),
        scratch_types=[
            pltpu.VMEM((_BPW,), jnp.int32),
            pltpu.VMEM((_BPW,), jnp.int32),
            pltpu.VMEM((_BPW,), jnp.int32),
            pltpu.VMEM((_BPW,), jnp.int32),
            pltpu.VMEM((_CHUNK, 4 * _D), jnp.float32),
            pltpu.VMEM((_CHUNK, 4 * _D), jnp.float32),
            pltpu.VMEM((_BPW,), jnp.float32),
            pltpu.SemaphoreType.DMA,
            pltpu.SemaphoreType.DMA,
        ],
    )(_mf_body)
    return run(u, v, ue2, ve2)
